# baseline (speedup 1.0000x reference)
import functools

import jax
import jax.numpy as jnp
from jax import lax
from jax.experimental import pallas as pl
from jax.experimental.pallas import tpu as pltpu
from jax.experimental.pallas import tpu_sc as plsc

B = 1024
V = 100000
S = 50
KS = (1, 5, 10, 20, 50)

NW = 32
RPW = B // NW
SPW = RPW * S
NCH = (SPW + 127) // 128
PADW = NCH * 128

RB = 1000
NBT = V // RB


def _tiled_flat_idx(r, c):
  return ((lax.shift_right_logical(c, 3) * 8192)
          + (lax.shift_right_logical(r, 7) * 1024)
          + ((c & 7) * 128) + (r & 127))


def _sc_gather(scores_flat, labels_flat, seqs_flat):
  mesh = plsc.VectorSubcoreMesh(core_axis_name="c", subcore_axis_name="s")

  @functools.partial(
      pl.kernel,
      mesh=mesh,
      out_type=(
          jax.ShapeDtypeStruct((B,), jnp.float32),
          jax.ShapeDtypeStruct((NW * PADW,), jnp.float32),
      ),
      scratch_types=[
          pltpu.VMEM((RPW,), jnp.int32),
          pltpu.VMEM((RPW,), jnp.int32),
          pltpu.VMEM((RPW,), jnp.float32),
          pltpu.VMEM((SPW,), jnp.int32),
          pltpu.VMEM((PADW,), jnp.int32),
          pltpu.VMEM((PADW,), jnp.float32),
          pltpu.SemaphoreType.DMA,
      ],
  )
  def k(scores_hbm, labels_hbm, seqs_hbm, pred_hbm, sval_hbm,
        lab_v, lidx_v, pred_v, seq_v, sidx_v, sval_v, sem):
    wid = lax.axis_index("s") * 2 + lax.axis_index("c")
    base = wid * RPW
    iota = lax.broadcasted_iota(jnp.int32, (16,), 0)

    pltpu.sync_copy(labels_hbm.at[pl.ds(base, RPW)], lab_v)
    for c in range(RPW // 16):
      rows = base + c * 16 + iota
      lidx_v[pl.ds(c * 16, 16)] = _tiled_flat_idx(rows, lab_v[pl.ds(c * 16, 16)])
    pltpu.async_copy(scores_hbm.at[lidx_v], pred_v, sem).wait()
    pltpu.sync_copy(pred_v, pred_hbm.at[pl.ds(base, RPW)])

    pltpu.sync_copy(seqs_hbm.at[pl.ds(base * S, SPW)], seq_v)
    for p in range(PADW // 16):
      if p * 16 < SPW:
        t = p * 16 + iota
        r_lo = (p * 16) // S
        r = r_lo + jnp.where(t >= (r_lo + 1) * S, 1, 0)
        sidx_v[pl.ds(p * 16, 16)] = _tiled_flat_idx(
            base + r, seq_v[pl.ds(p * 16, 16)])
      else:
        sidx_v[pl.ds(p * 16, 16)] = jnp.zeros((16,), jnp.int32) + base
    copies = [
        pltpu.async_copy(scores_hbm.at[sidx_v.at[pl.ds(c * 128, 128)]],
                         sval_v.at[pl.ds(c * 128, 128)], sem)
        for c in range(NCH)
    ]
    for cp in copies:
      cp.wait()
    pltpu.sync_copy(sval_v, sval_hbm.at[pl.ds(wid * PADW, PADW)])

  return k(scores_flat, labels_flat, seqs_flat)


def _tc_count(scores_t, pred_row, sval_t, seqs_t):
  def body(scores_ref, pred_ref, sval_ref, seqs_ref, out_ref, acc_ref):
    j = pl.program_id(0)

    @pl.when(j == 0)
    def _():
      acc_ref[...] = jnp.zeros_like(acc_ref)

    pred = pred_ref[...]
    blk = scores_ref[...]
    g = jnp.where(blk > jnp.broadcast_to(pred, (RB, B)), 1.0, 0.0)
    acc_ref[...] += jnp.sum(g.reshape(RB // 8, 8, B), axis=0)

    @pl.when(j == NBT - 1)
    def _():
      total = jnp.sum(acc_ref[...], axis=0, keepdims=True)
      seq = seqs_ref[...]
      sval = sval_ref[...]
      sub = jnp.zeros((1, B), jnp.float32)
      for t in range(S):
        gt_t = sval[t:t + 1, :] > pred
        if t > 0:
          dup = jnp.any(seq[:t, :] == seq[t:t + 1, :], axis=0, keepdims=True)
          gt_t = jnp.logical_and(gt_t, jnp.logical_not(dup))
        sub += jnp.where(gt_t, 1.0, 0.0)
      rank = total - sub
      inv_log = 1.0 / jnp.log2(rank + 2.0)
      row = lax.broadcasted_iota(jnp.int32, (8, 128), 0)
      colo = lax.broadcasted_iota(jnp.int32, (8, 128), 1)
      acc = jnp.zeros((8, 128), jnp.float32)
      for i, kk in enumerate(KS):
        ind = jnp.where(rank < float(kk), 1.0, 0.0)
        ndcg = jnp.sum(ind * inv_log) / B
        rec = jnp.sum(ind) / B
        acc = jnp.where((row == 0) & (colo == 2 * i), ndcg, acc)
        acc = jnp.where((row == 0) & (colo == 2 * i + 1), rec, acc)
      out_ref[...] = acc

  return pl.pallas_call(
      body,
      grid=(NBT,),
      in_specs=[
          pl.BlockSpec((RB, B), lambda j: (j, 0)),
          pl.BlockSpec((1, B), lambda j: (0, 0)),
          pl.BlockSpec((S, B), lambda j: (0, 0)),
          pl.BlockSpec((S, B), lambda j: (0, 0)),
      ],
      out_specs=pl.BlockSpec((8, 128), lambda j: (0, 0)),
      out_shape=jax.ShapeDtypeStruct((8, 128), jnp.float32),
      scratch_shapes=[pltpu.VMEM((8, B), jnp.float32)],
  )(scores_t, pred_row, sval_t, seqs_t)


def kernel(scores, labels, seqs):
  labels_i = labels.reshape(-1).astype(jnp.int32)
  seqs_i = seqs.astype(jnp.int32)
  scores_tiled_flat = (scores.T.reshape(V // 8, 8, B // 128, 128)
                       .transpose(0, 2, 1, 3).reshape(-1))
  pred, sval_pad = _sc_gather(scores_tiled_flat, labels_i,
                              seqs_i.reshape(-1))
  sval = sval_pad.reshape(NW, PADW)[:, :SPW].reshape(B, S)
  out = _tc_count(scores.T, pred.reshape(1, B), sval.T, seqs_i.T)
  return out[0, :10]


# speedup vs baseline: 1.1283x; 1.1283x over previous
import functools

import jax
import jax.numpy as jnp
from jax import lax
from jax.experimental import pallas as pl
from jax.experimental.pallas import tpu as pltpu
from jax.experimental.pallas import tpu_sc as plsc

B = 1024
V = 100000
S = 50
KS = (1, 5, 10, 20, 50)

NW = 32
RPW = B // NW
SPW = RPW * S
NCH = (SPW + 127) // 128
PADW = NCH * 128

RB = 2000
NBT = V // RB


def _tiled_flat_idx(r, c):
  return ((lax.shift_right_logical(c, 3) * 8192)
          + (lax.shift_right_logical(r, 7) * 1024)
          + ((c & 7) * 128) + (r & 127))


def _sc_gather(scores_flat, labels_flat, seqs_flat):
  mesh = plsc.VectorSubcoreMesh(core_axis_name="c", subcore_axis_name="s")

  @functools.partial(
      pl.kernel,
      mesh=mesh,
      out_type=(
          jax.ShapeDtypeStruct((B,), jnp.float32),
          jax.ShapeDtypeStruct((NW * PADW,), jnp.float32),
      ),
      scratch_types=[
          pltpu.VMEM((RPW,), jnp.int32),
          pltpu.VMEM((RPW,), jnp.int32),
          pltpu.VMEM((RPW,), jnp.float32),
          pltpu.VMEM((SPW,), jnp.int32),
          pltpu.VMEM((PADW,), jnp.int32),
          pltpu.VMEM((PADW,), jnp.float32),
          pltpu.SemaphoreType.DMA,
      ],
  )
  def k(scores_hbm, labels_hbm, seqs_hbm, pred_hbm, sval_hbm,
        lab_v, lidx_v, pred_v, seq_v, sidx_v, sval_v, sem):
    wid = lax.axis_index("s") * 2 + lax.axis_index("c")
    base = wid * RPW
    iota = lax.broadcasted_iota(jnp.int32, (16,), 0)

    pltpu.sync_copy(labels_hbm.at[pl.ds(base, RPW)], lab_v)
    for c in range(RPW // 16):
      rows = base + c * 16 + iota
      lidx_v[pl.ds(c * 16, 16)] = _tiled_flat_idx(rows, lab_v[pl.ds(c * 16, 16)])
    pltpu.async_copy(scores_hbm.at[lidx_v], pred_v, sem).wait()
    pltpu.sync_copy(pred_v, pred_hbm.at[pl.ds(base, RPW)])

    pltpu.sync_copy(seqs_hbm.at[pl.ds(base * S, SPW)], seq_v)
    for p in range(PADW // 16):
      if p * 16 < SPW:
        t = p * 16 + iota
        r_lo = (p * 16) // S
        r = r_lo + jnp.where(t >= (r_lo + 1) * S, 1, 0)
        sidx_v[pl.ds(p * 16, 16)] = _tiled_flat_idx(
            base + r, seq_v[pl.ds(p * 16, 16)])
      else:
        sidx_v[pl.ds(p * 16, 16)] = jnp.zeros((16,), jnp.int32) + base
    copies = [
        pltpu.async_copy(scores_hbm.at[sidx_v.at[pl.ds(c * 128, 128)]],
                         sval_v.at[pl.ds(c * 128, 128)], sem)
        for c in range(NCH)
    ]
    for cp in copies:
      cp.wait()
    pltpu.sync_copy(sval_v, sval_hbm.at[pl.ds(wid * PADW, PADW)])

  return k(scores_flat, labels_flat, seqs_flat)


def _tc_count(scores_t, pred_row, sval_t, seqs_t):
  def body(scores_ref, pred_ref, sval_ref, seqs_ref, out_ref, acc_ref):
    j = pl.program_id(0)

    @pl.when(j == 0)
    def _():
      acc_ref[...] = jnp.zeros_like(acc_ref)

    pred = pred_ref[...]
    blk = scores_ref[...]
    g = jnp.where(blk > jnp.broadcast_to(pred, (RB, B)), 1.0, 0.0)
    acc_ref[...] += jnp.sum(g.reshape(RB // 8, 8, B), axis=0)

    @pl.when(j == NBT - 1)
    def _():
      total = jnp.sum(acc_ref[...], axis=0, keepdims=True)
      seq = seqs_ref[...]
      sval = sval_ref[...]
      sub = jnp.zeros((1, B), jnp.float32)
      for t in range(S):
        gt_t = sval[t:t + 1, :] > pred
        if t > 0:
          dup = jnp.any(seq[:t, :] == seq[t:t + 1, :], axis=0, keepdims=True)
          gt_t = jnp.logical_and(gt_t, jnp.logical_not(dup))
        sub += jnp.where(gt_t, 1.0, 0.0)
      rank = total - sub
      inv_log = 1.0 / jnp.log2(rank + 2.0)
      row = lax.broadcasted_iota(jnp.int32, (8, 128), 0)
      colo = lax.broadcasted_iota(jnp.int32, (8, 128), 1)
      acc = jnp.zeros((8, 128), jnp.float32)
      for i, kk in enumerate(KS):
        ind = jnp.where(rank < float(kk), 1.0, 0.0)
        ndcg = jnp.sum(ind * inv_log) / B
        rec = jnp.sum(ind) / B
        acc = jnp.where((row == 0) & (colo == 2 * i), ndcg, acc)
        acc = jnp.where((row == 0) & (colo == 2 * i + 1), rec, acc)
      out_ref[...] = acc

  return pl.pallas_call(
      body,
      grid=(NBT,),
      in_specs=[
          pl.BlockSpec((RB, B), lambda j: (j, 0)),
          pl.BlockSpec((1, B), lambda j: (0, 0)),
          pl.BlockSpec((S, B), lambda j: (0, 0)),
          pl.BlockSpec((S, B), lambda j: (0, 0)),
      ],
      out_specs=pl.BlockSpec((8, 128), lambda j: (0, 0)),
      out_shape=jax.ShapeDtypeStruct((8, 128), jnp.float32),
      scratch_shapes=[pltpu.VMEM((8, B), jnp.float32)],
  )(scores_t, pred_row, sval_t, seqs_t)


def kernel(scores, labels, seqs):
  labels_i = labels.reshape(-1).astype(jnp.int32)
  seqs_i = seqs.astype(jnp.int32)
  scores_tiled_flat = (scores.T.reshape(V // 8, 8, B // 128, 128)
                       .transpose(0, 2, 1, 3).reshape(-1))
  pred, sval_pad = _sc_gather(scores_tiled_flat, labels_i,
                              seqs_i.reshape(-1))
  sval = sval_pad.reshape(NW, PADW)[:, :SPW].reshape(B, S)
  out = _tc_count(scores.T, pred.reshape(1, B), sval.T, seqs_i.T)
  return out[0, :10]
